# baseline (speedup 1.0000x reference)
import dataclasses
import functools

import jax
import jax.numpy as jnp
from jax import lax
from jax.experimental import pallas as pl
from jax.experimental.pallas import tpu as pltpu
from jax.experimental.pallas import tpu_sc as plsc

NC = 2
NS = 16
NW = NC * NS
D = 128
W = 64
NB = 4
K = 2
CH = 16
N_PAD = 10240
RPT = N_PAD // NS


def _vmesh():
    return plsc.VectorSubcoreMesh(core_axis_name="c", subcore_axis_name="s")


def _no_layout_params():
    cp = pltpu.CompilerParams()
    if "needs_layout_passes" in pltpu.CompilerParams.__dataclass_fields__:
        cp = dataclasses.replace(cp, needs_layout_passes=False)
    return cp


def _deg_partials(row_idx):
    epb = row_idx.shape[2]

    @functools.partial(
        pl.kernel,
        out_type=jax.ShapeDtypeStruct((NC, N_PAD), jnp.int32),
        mesh=_vmesh(),
        compiler_params=_no_layout_params(),
        scratch_types=[
            pltpu.VMEM((epb,), jnp.int32),
            pltpu.VMEM((N_PAD,), jnp.int32),
            pltpu.VMEM((NS, RPT), jnp.int32),
            pltpu.VMEM_SHARED((NS, N_PAD), jnp.int32),
        ],
    )
    def deg_kernel(row_hbm, out_hbm, idx_v, hist, red, shared):
        c = lax.axis_index("c")
        s = lax.axis_index("s")

        pltpu.sync_copy(row_hbm.at[c, s], idx_v)

        @pl.loop(0, N_PAD, step=16)
        def _(i):
            hist.at[pl.ds(i, 16)][...] = jnp.zeros((16,), jnp.int32)

        @pl.loop(0, epb, step=16)
        def _(k):
            idx = idx_v.at[pl.ds(k, 16)][...]
            cnt, last = plsc.scan_count(idx)
            plsc.addupdate_scatter(hist, [idx], cnt, mask=last)

        pltpu.sync_copy(hist, shared.at[s])
        plsc.subcore_barrier()

        for r in range(NS):
            pltpu.sync_copy(shared.at[r, pl.ds(s * RPT, RPT)],
                            red.at[r, pl.ds(0, RPT)])

        @pl.loop(0, RPT, step=16)
        def _(k):
            acc = red.at[0, pl.ds(k, 16)][...]
            for r in range(1, NS):
                acc = acc + red.at[r, pl.ds(k, 16)][...]
            hist.at[pl.ds(k, 16)][...] = acc

        pltpu.sync_copy(hist.at[pl.ds(0, RPT)],
                        out_hbm.at[c, pl.ds(s * RPT, RPT)])

    return deg_kernel(row_idx)


def _hop_partials(u, col_idx, row_idx):
    win = col_idx.shape[2]
    nchunk = win // CH
    assert win % CH == 0 and CH % NB == 0 and K <= NB
    col_idx = col_idx.reshape(NC, NS, nchunk, CH, W)
    row_idx = row_idx.reshape(NC, NS, nchunk, CH, W)

    @functools.partial(
        pl.kernel,
        out_type=jax.ShapeDtypeStruct((NC, N_PAD, D), jnp.float32),
        mesh=_vmesh(),
        scratch_types=[
            pltpu.VMEM((2, CH, W), jnp.int32),
            pltpu.VMEM((2, CH, W), jnp.int32),
            pltpu.VMEM((NB, W, D), jnp.float32),
            pltpu.VMEM_SHARED((N_PAD, D), jnp.float32),
            pltpu.SemaphoreType.DMA((NB,)),
            pltpu.SemaphoreType.DMA((NB,)),
            pltpu.SemaphoreType.DMA,
        ],
    )
    def hop_kernel(u_hbm, col_hbm, row_hbm, out_hbm,
                   col_v, row_v, bufs, acc, gsem, ssem, isem):
        c = lax.axis_index("c")
        s = lax.axis_index("s")

        @pl.loop(0, W)
        def _(i):
            @pl.loop(0, D, step=16)
            def _(kk):
                bufs.at[0, i, pl.ds(kk, 16)][...] = jnp.zeros((16,), jnp.float32)

        @pl.loop(0, RPT, step=W)
        def _(r):
            pltpu.async_copy(bufs.at[0], acc.at[pl.ds(s * RPT + r, W)], isem)

        pltpu.sync_copy(col_hbm.at[c, s, 0], col_v.at[0])
        pltpu.sync_copy(row_hbm.at[c, s, 0], row_v.at[0])

        @pl.loop(0, RPT, step=W)
        def _(r):
            pltpu.make_async_copy(bufs.at[0], acc.at[pl.ds(0, W)], isem).wait()

        plsc.subcore_barrier()

        for b in range(K):
            pltpu.async_copy(u_hbm.at[col_v.at[0, b]], bufs.at[b], gsem.at[b])

        @pl.loop(0, nchunk)
        def _(ci):
            slot = jnp.bitwise_and(ci, 1)
            nslot = jnp.bitwise_and(ci + 1, 1)
            not_last = ci + 1 < nchunk

            @pl.loop(0, CH, step=NB)
            def _(l0):
                @pl.when(jnp.logical_and(l0 == NB, not_last))
                def _():
                    pltpu.async_copy(col_hbm.at[c, s, ci + 1],
                                     col_v.at[nslot], isem)
                    pltpu.async_copy(row_hbm.at[c, s, ci + 1],
                                     row_v.at[nslot], isem)

                @pl.when(jnp.logical_and(l0 == CH - NB, not_last))
                def _():
                    pltpu.make_async_copy(col_hbm.at[c, s, 0],
                                          col_v.at[nslot], isem).wait()
                    pltpu.make_async_copy(row_hbm.at[c, s, 0],
                                          row_v.at[nslot], isem).wait()

                for b in range(NB):
                    l = l0 + b
                    pltpu.make_async_copy(u_hbm.at[col_v.at[0, 0]],
                                          bufs.at[b], gsem.at[b]).wait()
                    pltpu.async_copy(bufs.at[b], acc.at[row_v.at[slot, l]],
                                     ssem.at[b], add=True)
                    lf = l + K
                    bf = (b + K) % NB
                    crossed = lf >= CH
                    lf2 = lax.select(crossed, lf - CH, lf)
                    slot_f = lax.select(crossed, nslot, slot)

                    @pl.when(jnp.logical_or(not_last, jnp.logical_not(crossed)))
                    def _():
                        @pl.when(jnp.logical_or(ci > 0, lf >= NB))
                        def _():
                            pltpu.make_async_copy(bufs.at[bf],
                                                  acc.at[row_v.at[0, 0]],
                                                  ssem.at[bf]).wait()
                        pltpu.async_copy(u_hbm.at[col_v.at[slot_f, lf2]],
                                         bufs.at[bf], gsem.at[bf])

        for b in range(NB):
            pltpu.make_async_copy(bufs.at[b], acc.at[row_v.at[0, 0]],
                                  ssem.at[b]).wait()

        plsc.subcore_barrier()
        pltpu.sync_copy(acc.at[pl.ds(s * RPT, RPT)],
                        out_hbm.at[c, pl.ds(s * RPT, RPT)])

    return hop_kernel(u, col_idx, row_idx)


def _matmul(x, w):
    m, k = x.shape
    nout = w.shape[1]
    bm = 1280
    assert m % bm == 0

    def body(x_ref, w_ref, o_ref):
        o_ref[...] = jax.lax.dot(x_ref[...], w_ref[...],
                                 precision=jax.lax.Precision.HIGHEST,
                                 preferred_element_type=jnp.float32)

    return pl.pallas_call(
        body,
        grid=(m // bm,),
        in_specs=[
            pl.BlockSpec((bm, k), lambda i: (i, 0)),
            pl.BlockSpec((k, nout), lambda i: (0, 0)),
        ],
        out_specs=pl.BlockSpec((bm, nout), lambda i: (i, 0)),
        out_shape=jax.ShapeDtypeStruct((m, nout), jnp.float32),
    )(x, w)


def kernel(x, edge_index, kernel, bias):
    w = kernel
    n = x.shape[0]
    e = edge_index.shape[1]
    epb = e // NW
    epb_pad = -(-epb // (W * CH)) * (W * CH)
    win = epb_pad // W
    assert e % NW == 0 and win % CH == 0 and n < N_PAD

    row3 = edge_index[0].astype(jnp.int32).reshape(NC, NS, epb)
    col3 = edge_index[1].astype(jnp.int32).reshape(NC, NS, epb)
    pad = epb_pad - epb
    pad_dst = n + (jnp.arange(pad, dtype=jnp.int32) % (N_PAD - n))
    pad_src = jnp.arange(pad, dtype=jnp.int32) % n
    rowp = jnp.concatenate(
        [row3, jnp.broadcast_to(pad_dst, (NC, NS, pad))], axis=2)
    colp = jnp.concatenate(
        [col3, jnp.broadcast_to(pad_src, (NC, NS, pad))], axis=2)
    row = rowp.reshape(NC, NS, win, W)
    col = colp.reshape(NC, NS, win, W)

    degp = _deg_partials(row3)
    deg = (degp[0] + degp[1] + 1).astype(jnp.float32)
    dis = lax.rsqrt(deg)[:, None]
    dinv = (1.0 / deg)[:, None]

    xp = jnp.concatenate(
        [x, jnp.zeros((N_PAD - n, x.shape[1]), x.dtype)], axis=0)
    u0 = _matmul(xp, w) * dis
    p1 = _hop_partials(u0, col, row)
    u1 = (p1[0] + p1[1] + u0) * dinv
    p2 = _hop_partials(u1, col, row)
    out = (p2[0] + p2[1] + u1) * dis + bias[None, :]
    return out[:n]


# speedup vs baseline: 1.1300x; 1.1300x over previous
import dataclasses
import functools

import jax
import jax.numpy as jnp
from jax import lax
from jax.experimental import pallas as pl
from jax.experimental.pallas import tpu as pltpu
from jax.experimental.pallas import tpu_sc as plsc

NC = 2
NS = 16
NW = NC * NS
D = 128
W = 64
NB = 4
K = 3
CH = 16
N_PAD = 10240
RPT = N_PAD // NS


def _vmesh():
    return plsc.VectorSubcoreMesh(core_axis_name="c", subcore_axis_name="s")


def _no_layout_params():
    cp = pltpu.CompilerParams()
    if "needs_layout_passes" in pltpu.CompilerParams.__dataclass_fields__:
        cp = dataclasses.replace(cp, needs_layout_passes=False)
    return cp


def _deg_partials(row_idx):
    epb = row_idx.shape[2]

    @functools.partial(
        pl.kernel,
        out_type=jax.ShapeDtypeStruct((NC, N_PAD), jnp.int32),
        mesh=_vmesh(),
        compiler_params=_no_layout_params(),
        scratch_types=[
            pltpu.VMEM((epb,), jnp.int32),
            pltpu.VMEM((N_PAD,), jnp.int32),
            pltpu.VMEM((NS, RPT), jnp.int32),
            pltpu.VMEM_SHARED((NS, N_PAD), jnp.int32),
        ],
    )
    def deg_kernel(row_hbm, out_hbm, idx_v, hist, red, shared):
        c = lax.axis_index("c")
        s = lax.axis_index("s")

        pltpu.sync_copy(row_hbm.at[c, s], idx_v)

        @pl.loop(0, N_PAD, step=16)
        def _(i):
            hist.at[pl.ds(i, 16)][...] = jnp.zeros((16,), jnp.int32)

        @pl.loop(0, epb, step=16)
        def _(k):
            idx = idx_v.at[pl.ds(k, 16)][...]
            cnt, last = plsc.scan_count(idx)
            plsc.addupdate_scatter(hist, [idx], cnt, mask=last)

        pltpu.sync_copy(hist, shared.at[s])
        plsc.subcore_barrier()

        for r in range(NS):
            pltpu.sync_copy(shared.at[r, pl.ds(s * RPT, RPT)],
                            red.at[r, pl.ds(0, RPT)])

        @pl.loop(0, RPT, step=16)
        def _(k):
            acc = red.at[0, pl.ds(k, 16)][...]
            for r in range(1, NS):
                acc = acc + red.at[r, pl.ds(k, 16)][...]
            hist.at[pl.ds(k, 16)][...] = acc

        pltpu.sync_copy(hist.at[pl.ds(0, RPT)],
                        out_hbm.at[c, pl.ds(s * RPT, RPT)])

    return deg_kernel(row_idx)


def _hop_partials(u, col_idx, row_idx):
    win = col_idx.shape[2]
    nchunk = win // CH
    assert win % CH == 0 and CH % NB == 0 and K <= NB
    col_idx = col_idx.reshape(NC, NS, nchunk, CH, W)
    row_idx = row_idx.reshape(NC, NS, nchunk, CH, W)

    @functools.partial(
        pl.kernel,
        out_type=jax.ShapeDtypeStruct((NC, N_PAD, D), jnp.float32),
        mesh=_vmesh(),
        scratch_types=[
            pltpu.VMEM((2, CH, W), jnp.int32),
            pltpu.VMEM((2, CH, W), jnp.int32),
            pltpu.VMEM((NB, W, D), jnp.float32),
            pltpu.VMEM_SHARED((N_PAD, D), jnp.float32),
            pltpu.SemaphoreType.DMA((NB,)),
            pltpu.SemaphoreType.DMA((NB,)),
            pltpu.SemaphoreType.DMA,
        ],
    )
    def hop_kernel(u_hbm, col_hbm, row_hbm, out_hbm,
                   col_v, row_v, bufs, acc, gsem, ssem, isem):
        c = lax.axis_index("c")
        s = lax.axis_index("s")

        @pl.loop(0, W)
        def _(i):
            @pl.loop(0, D, step=16)
            def _(kk):
                bufs.at[0, i, pl.ds(kk, 16)][...] = jnp.zeros((16,), jnp.float32)

        @pl.loop(0, RPT, step=W)
        def _(r):
            pltpu.async_copy(bufs.at[0], acc.at[pl.ds(s * RPT + r, W)], isem)

        pltpu.sync_copy(col_hbm.at[c, s, 0], col_v.at[0])
        pltpu.sync_copy(row_hbm.at[c, s, 0], row_v.at[0])

        @pl.loop(0, RPT, step=W)
        def _(r):
            pltpu.make_async_copy(bufs.at[0], acc.at[pl.ds(0, W)], isem).wait()

        plsc.subcore_barrier()

        for b in range(K):
            pltpu.async_copy(u_hbm.at[col_v.at[0, b]], bufs.at[b], gsem.at[b])

        @pl.loop(0, nchunk)
        def _(ci):
            slot = jnp.bitwise_and(ci, 1)
            nslot = jnp.bitwise_and(ci + 1, 1)
            not_last = ci + 1 < nchunk

            @pl.loop(0, CH, step=NB)
            def _(l0):
                @pl.when(jnp.logical_and(l0 == NB, not_last))
                def _():
                    pltpu.async_copy(col_hbm.at[c, s, ci + 1],
                                     col_v.at[nslot], isem)
                    pltpu.async_copy(row_hbm.at[c, s, ci + 1],
                                     row_v.at[nslot], isem)

                @pl.when(jnp.logical_and(l0 == CH - NB, not_last))
                def _():
                    pltpu.make_async_copy(col_hbm.at[c, s, 0],
                                          col_v.at[nslot], isem).wait()
                    pltpu.make_async_copy(row_hbm.at[c, s, 0],
                                          row_v.at[nslot], isem).wait()

                for b in range(NB):
                    l = l0 + b
                    pltpu.make_async_copy(u_hbm.at[col_v.at[0, 0]],
                                          bufs.at[b], gsem.at[b]).wait()
                    pltpu.async_copy(bufs.at[b], acc.at[row_v.at[slot, l]],
                                     ssem.at[b], add=True)
                    lf = l + K
                    bf = (b + K) % NB
                    crossed = lf >= CH
                    lf2 = lax.select(crossed, lf - CH, lf)
                    slot_f = lax.select(crossed, nslot, slot)

                    @pl.when(jnp.logical_or(not_last, jnp.logical_not(crossed)))
                    def _():
                        @pl.when(jnp.logical_or(ci > 0, lf >= NB))
                        def _():
                            pltpu.make_async_copy(bufs.at[bf],
                                                  acc.at[row_v.at[0, 0]],
                                                  ssem.at[bf]).wait()
                        pltpu.async_copy(u_hbm.at[col_v.at[slot_f, lf2]],
                                         bufs.at[bf], gsem.at[bf])

        for b in range(NB):
            pltpu.make_async_copy(bufs.at[b], acc.at[row_v.at[0, 0]],
                                  ssem.at[b]).wait()

        plsc.subcore_barrier()
        pltpu.sync_copy(acc.at[pl.ds(s * RPT, RPT)],
                        out_hbm.at[c, pl.ds(s * RPT, RPT)])

    return hop_kernel(u, col_idx, row_idx)


def _matmul(x, w):
    m, k = x.shape
    nout = w.shape[1]
    bm = 2000
    assert m % bm == 0

    def body(x_ref, w_ref, o_ref):
        o_ref[...] = jax.lax.dot(x_ref[...], w_ref[...],
                                 precision=jax.lax.Precision.HIGHEST,
                                 preferred_element_type=jnp.float32)

    return pl.pallas_call(
        body,
        grid=(m // bm,),
        in_specs=[
            pl.BlockSpec((bm, k), lambda i: (i, 0)),
            pl.BlockSpec((k, nout), lambda i: (0, 0)),
        ],
        out_specs=pl.BlockSpec((bm, nout), lambda i: (i, 0)),
        out_shape=jax.ShapeDtypeStruct((m, nout), jnp.float32),
    )(x, w)


def kernel(x, edge_index, kernel, bias):
    w = kernel
    n = x.shape[0]
    e = edge_index.shape[1]
    epb = e // NW
    epb_pad = -(-epb // (W * CH)) * (W * CH)
    win = epb_pad // W
    assert e % NW == 0 and win % CH == 0 and n < N_PAD

    row3 = edge_index[0].astype(jnp.int32).reshape(NC, NS, epb)
    col3 = edge_index[1].astype(jnp.int32).reshape(NC, NS, epb)
    pad = epb_pad - epb
    pad_dst = n + (jnp.arange(pad, dtype=jnp.int32) % (N_PAD - n))
    pad_src = jnp.arange(pad, dtype=jnp.int32) % n
    rowp = jnp.concatenate(
        [row3, jnp.broadcast_to(pad_dst, (NC, NS, pad))], axis=2)
    colp = jnp.concatenate(
        [col3, jnp.broadcast_to(pad_src, (NC, NS, pad))], axis=2)
    row = rowp.reshape(NC, NS, win, W)
    col = colp.reshape(NC, NS, win, W)

    degp = _deg_partials(row3)
    deg = (degp[0] + degp[1] + 1).astype(jnp.float32)
    dis = lax.rsqrt(deg)[:, None]
    dinv = (1.0 / deg)[:, None]

    u0 = _matmul(x, w) * dis[:n]
    p1 = _hop_partials(u0, col, row)
    u1 = (p1[0, :n] + p1[1, :n] + u0) * dinv[:n]
    p2 = _hop_partials(u1, col, row)
    return (p2[0, :n] + p2[1, :n] + u1) * dis[:n] + bias[None, :]
